# baseline (speedup 1.0000x reference)
import functools

import jax
import jax.numpy as jnp
from jax import lax
from jax.experimental import pallas as pl
from jax.experimental.pallas import tpu as pltpu
from jax.experimental.pallas import tpu_sc as plsc

H = 768
F = 1024
E = 64
K = 2
S = 2048
P = S * K
BLK = 128
LOG2BLK = 7
G = 96
NSLOT = G * BLK
NEG = -1e30

_MESH = plsc.VectorSubcoreMesh(core_axis_name="c", subcore_axis_name="s")
_SC_PARAMS = pltpu.CompilerParams(needs_layout_passes=False)


def _router_body(x_ref, gw_ref, topi_ref, topw_ref):
    logits = jnp.dot(x_ref[...], gw_ref[...], preferred_element_type=jnp.float32)
    idx = lax.broadcasted_iota(jnp.int32, (S, E), 1)
    m1 = jnp.max(logits, axis=1, keepdims=True)
    a1 = jnp.min(jnp.where(logits >= m1, idx, E), axis=1, keepdims=True)
    l2 = jnp.where(idx == a1, NEG, logits)
    m2 = jnp.max(l2, axis=1, keepdims=True)
    a2 = jnp.min(jnp.where((l2 >= m2) & (idx != a1), idx, E), axis=1, keepdims=True)
    e2 = jnp.exp(m2 - m1)
    w1 = 1.0 / (1.0 + e2)
    topi_ref[...] = jnp.concatenate([a1, a2], axis=1)
    topw_ref[...] = jnp.concatenate([w1, 1.0 - w1], axis=1)


def _router(x, gate_w):
    return pl.pallas_call(
        _router_body,
        out_shape=(
            jax.ShapeDtypeStruct((S, K), jnp.int32),
            jax.ShapeDtypeStruct((S, K), jnp.float32),
        ),
    )(x, gate_w)


def _iota16():
    return lax.broadcasted_iota(jnp.int32, (16,), 0)


@functools.partial(
    pl.kernel,
    mesh=_MESH,
    out_type=(
        jax.ShapeDtypeStruct((NSLOT,), jnp.int32),
        jax.ShapeDtypeStruct((NSLOT,), jnp.float32),
        jax.ShapeDtypeStruct((G,), jnp.int32),
        jax.ShapeDtypeStruct((G,), jnp.int32),
    ),
    scratch_types=(
        pltpu.VMEM((P,), jnp.int32),
        pltpu.VMEM((P,), jnp.int32),
        pltpu.VMEM((E,), jnp.int32),
        pltpu.VMEM((E,), jnp.int32),
        pltpu.VMEM((112,), jnp.int32),
        pltpu.VMEM((NSLOT,), jnp.int32),
        pltpu.VMEM((P,), jnp.float32),
        pltpu.VMEM((NSLOT,), jnp.float32),
        pltpu.VMEM((G,), jnp.int32),
        pltpu.VMEM((G,), jnp.int32),
    ),
    compiler_params=_SC_PARAMS,
)
def _plan(eid_hbm, wp_hbm, src_hbm, ws_hbm, be_hbm, bv_hbm,
          eid_v, rank_v, cnt_v, po_v, eh_v, src_v, wp_v, ws_v, be_v, bv_v):
    wid = lax.axis_index("s") * 2 + lax.axis_index("c")

    @pl.when(wid == 0)
    def _():
        pltpu.sync_copy(eid_hbm, eid_v)
        pltpu.sync_copy(wp_hbm, wp_v)
        zeros = jnp.zeros((16,), jnp.int32)
        for g in range(E // 16):
            cnt_v[pl.ds(g * 16, 16)] = zeros

        def rank_body(g, c):
            v = eid_v[pl.ds(g * 16, 16)]
            base = plsc.load_gather(cnt_v, (v,))
            dup, lastm = plsc.scan_count(v)
            rank_v[pl.ds(g * 16, 16)] = base + dup - 1
            plsc.store_scatter(cnt_v, (v,), base + dup, mask=lastm)
            return c

        lax.fori_loop(0, P // 16, rank_body, 0)

        for g in range(7):
            eh_v[pl.ds(g * 16, 16)] = zeros
        ones = jnp.ones((16,), jnp.int32)
        carry = jnp.int32(0)
        e_last = jnp.int32(0)
        for g in range(E // 16):
            cnt = cnt_v[pl.ds(g * 16, 16)]
            nb = (cnt + (BLK - 1)) >> LOG2BLK
            incl = plsc.cumsum(nb)
            excl = carry + incl - nb
            po_v[pl.ds(g * 16, 16)] = excl * BLK
            plsc.addupdate_scatter(eh_v, (excl + nb,), ones)
            carry = carry + jnp.max(incl, axis=0)
            gid = _iota16() + g * 16
            e_last = jnp.maximum(e_last, jnp.max(jnp.where(cnt > 0, gid, -1), axis=0))
        nblocks = carry

        bcarry = jnp.int32(0)
        for g in range(G // 16):
            h = eh_v[pl.ds(g * 16, 16)]
            c = plsc.cumsum(h) + bcarry
            bid = _iota16() + g * 16
            valid = bid < nblocks
            be_v[pl.ds(g * 16, 16)] = jnp.where(valid, jnp.minimum(c, E - 1), e_last)
            bv_v[pl.ds(g * 16, 16)] = jnp.where(valid, 1, 0)
            bcarry = jnp.max(c, axis=0)

        fzeros = jnp.zeros((16,), jnp.float32)

        def zero_body(g, c):
            ws_v[pl.ds(g * 16, 16)] = fzeros
            return c

        lax.fori_loop(0, NSLOT // 16, zero_body, 0)

        def dest_body(g, c):
            v = eid_v[pl.ds(g * 16, 16)]
            d = plsc.load_gather(po_v, (v,)) + rank_v[pl.ds(g * 16, 16)]
            tok = (_iota16() + g * 16) >> 1
            plsc.store_scatter(src_v, (d,), tok)
            plsc.store_scatter(ws_v, (d,), wp_v[pl.ds(g * 16, 16)])
            return c

        lax.fori_loop(0, P // 16, dest_body, 0)

        pltpu.sync_copy(src_v, src_hbm)
        pltpu.sync_copy(ws_v, ws_hbm)
        pltpu.sync_copy(be_v, be_hbm)
        pltpu.sync_copy(bv_v, bv_hbm)


def _ffn_body(be_ref, bv_ref, srcc_ref, srcr_ref, wsc_ref, x_ref,
              w1_ref, w3_ref, w2_ref, out_ref):
    b = pl.program_id(0)

    @pl.when(b == 0)
    def _():
        out_ref[...] = jnp.zeros_like(out_ref)

    @pl.when(bv_ref[b] != 0)
    def _():
        tcol = srcc_ref[...]
        iota1 = lax.broadcasted_iota(jnp.int32, (BLK, S), 1)
        sel = (tcol == iota1).astype(jnp.bfloat16)
        xb = jnp.dot(sel, x_ref[...], preferred_element_type=jnp.float32)
        xb = xb.astype(jnp.bfloat16)
        g = jnp.dot(xb, w1_ref[0].astype(jnp.bfloat16),
                    preferred_element_type=jnp.float32)
        u = jnp.dot(xb, w3_ref[0].astype(jnp.bfloat16),
                    preferred_element_type=jnp.float32)
        act = g * (1.0 / (1.0 + jnp.exp(-g)))
        part = jnp.dot((act * u).astype(jnp.bfloat16),
                       w2_ref[0].astype(jnp.bfloat16),
                       preferred_element_type=jnp.float32)
        yw = (part * wsc_ref[...]).astype(jnp.bfloat16)
        srow = srcr_ref[0]
        iota0 = lax.broadcasted_iota(jnp.int32, (S, BLK), 0)
        selT = (iota0 == srow).astype(jnp.bfloat16)
        out_ref[...] += jnp.dot(selT, yw, preferred_element_type=jnp.float32)


def _ffn(be, bv, src_col, src_row, ws_col, x, w1, w3, w2):
    grid_spec = pltpu.PrefetchScalarGridSpec(
        num_scalar_prefetch=2,
        grid=(G,),
        in_specs=[
            pl.BlockSpec((BLK, 1), lambda b, be, bv: (b, 0)),
            pl.BlockSpec((1, 1, BLK), lambda b, be, bv: (b, 0, 0)),
            pl.BlockSpec((BLK, 1), lambda b, be, bv: (b, 0)),
            pl.BlockSpec((S, H), lambda b, be, bv: (0, 0)),
            pl.BlockSpec((1, H, F), lambda b, be, bv: (be[b], 0, 0)),
            pl.BlockSpec((1, H, F), lambda b, be, bv: (be[b], 0, 0)),
            pl.BlockSpec((1, F, H), lambda b, be, bv: (be[b], 0, 0)),
        ],
        out_specs=pl.BlockSpec((S, H), lambda b, be, bv: (0, 0)),
    )
    x = x.astype(jnp.bfloat16)
    return pl.pallas_call(
        _ffn_body,
        grid_spec=grid_spec,
        out_shape=jax.ShapeDtypeStruct((S, H), jnp.float32),
        compiler_params=pltpu.CompilerParams(vmem_limit_bytes=117 * 1024 * 1024),
    )(be, bv, src_col, src_row, ws_col, x, w1, w3, w2)


def kernel(hidden_states, gate_w, w1, w2, w3):
    b, s, h = hidden_states.shape
    x = hidden_states.reshape(s, h)
    topi, topw = _router(x, gate_w)
    src, ws, be, bv = _plan(topi.reshape(-1), topw.reshape(-1))
    out = _ffn(be, bv, src.reshape(NSLOT, 1), src.reshape(G, 1, BLK),
               ws.reshape(NSLOT, 1), x, w1, w3, w2)
    return out.reshape(b, s, h)


# speedup vs baseline: 1.0379x; 1.0379x over previous
import functools

import jax
import jax.numpy as jnp
from jax import lax
from jax.experimental import pallas as pl
from jax.experimental.pallas import tpu as pltpu
from jax.experimental.pallas import tpu_sc as plsc

H = 768
F = 1024
E = 64
K = 2
S = 2048
P = S * K
BLK = 128
LOG2BLK = 7
G = 96
NSLOT = G * BLK
NEG = -1e30

_MESH = plsc.VectorSubcoreMesh(core_axis_name="c", subcore_axis_name="s")
_SC_PARAMS = pltpu.CompilerParams(needs_layout_passes=False)


def _router_body(x_ref, gw_ref, topi_ref, topw_ref):
    logits = jnp.dot(x_ref[...], gw_ref[...], preferred_element_type=jnp.float32)
    idx = lax.broadcasted_iota(jnp.int32, (S, E), 1)
    m1 = jnp.max(logits, axis=1, keepdims=True)
    a1 = jnp.min(jnp.where(logits >= m1, idx, E), axis=1, keepdims=True)
    l2 = jnp.where(idx == a1, NEG, logits)
    m2 = jnp.max(l2, axis=1, keepdims=True)
    a2 = jnp.min(jnp.where((l2 >= m2) & (idx != a1), idx, E), axis=1, keepdims=True)
    e2 = jnp.exp(m2 - m1)
    w1 = 1.0 / (1.0 + e2)
    topi_ref[...] = jnp.concatenate([a1, a2], axis=1)
    topw_ref[...] = jnp.concatenate([w1, 1.0 - w1], axis=1)


def _router(x, gate_w):
    return pl.pallas_call(
        _router_body,
        out_shape=(
            jax.ShapeDtypeStruct((S, K), jnp.int32),
            jax.ShapeDtypeStruct((S, K), jnp.float32),
        ),
    )(x, gate_w)


def _iota16():
    return lax.broadcasted_iota(jnp.int32, (16,), 0)


@functools.partial(
    pl.kernel,
    mesh=_MESH,
    out_type=(
        jax.ShapeDtypeStruct((NSLOT,), jnp.int32),
        jax.ShapeDtypeStruct((NSLOT,), jnp.float32),
        jax.ShapeDtypeStruct((G,), jnp.int32),
        jax.ShapeDtypeStruct((G,), jnp.int32),
    ),
    scratch_types=(
        pltpu.VMEM((P,), jnp.int32),
        pltpu.VMEM((P,), jnp.int32),
        pltpu.VMEM((E,), jnp.int32),
        pltpu.VMEM((E,), jnp.int32),
        pltpu.VMEM((112,), jnp.int32),
        pltpu.VMEM((NSLOT,), jnp.int32),
        pltpu.VMEM((P,), jnp.float32),
        pltpu.VMEM((NSLOT,), jnp.float32),
        pltpu.VMEM((G,), jnp.int32),
        pltpu.VMEM((G,), jnp.int32),
    ),
    compiler_params=_SC_PARAMS,
)
def _plan(eid_hbm, wp_hbm, src_hbm, ws_hbm, be_hbm, bv_hbm,
          eid_v, rank_v, cnt_v, po_v, eh_v, src_v, wp_v, ws_v, be_v, bv_v):
    wid = lax.axis_index("s") * 2 + lax.axis_index("c")

    @pl.when(wid == 0)
    def _():
        pltpu.sync_copy(eid_hbm, eid_v)
        pltpu.sync_copy(wp_hbm, wp_v)
        zeros = jnp.zeros((16,), jnp.int32)
        for g in range(E // 16):
            cnt_v[pl.ds(g * 16, 16)] = zeros

        def rank_body(g, c):
            v = eid_v[pl.ds(g * 16, 16)]
            base = plsc.load_gather(cnt_v, (v,))
            dup, lastm = plsc.scan_count(v)
            rank_v[pl.ds(g * 16, 16)] = base + dup - 1
            plsc.store_scatter(cnt_v, (v,), base + dup, mask=lastm)
            return c

        lax.fori_loop(0, P // 16, rank_body, 0)

        for g in range(7):
            eh_v[pl.ds(g * 16, 16)] = zeros
        ones = jnp.ones((16,), jnp.int32)
        carry = jnp.int32(0)
        e_last = jnp.int32(0)
        for g in range(E // 16):
            cnt = cnt_v[pl.ds(g * 16, 16)]
            nb = (cnt + (BLK - 1)) >> LOG2BLK
            incl = plsc.cumsum(nb)
            excl = carry + incl - nb
            po_v[pl.ds(g * 16, 16)] = excl * BLK
            plsc.addupdate_scatter(eh_v, (excl + nb,), ones)
            carry = carry + jnp.max(incl, axis=0)
            gid = _iota16() + g * 16
            e_last = jnp.maximum(e_last, jnp.max(jnp.where(cnt > 0, gid, -1), axis=0))
        nblocks = carry

        bcarry = jnp.int32(0)
        for g in range(G // 16):
            h = eh_v[pl.ds(g * 16, 16)]
            c = plsc.cumsum(h) + bcarry
            bid = _iota16() + g * 16
            valid = bid < nblocks
            be_v[pl.ds(g * 16, 16)] = jnp.where(valid, jnp.minimum(c, E - 1), e_last)
            bv_v[pl.ds(g * 16, 16)] = jnp.where(valid, 1, 0)
            bcarry = jnp.max(c, axis=0)

        fzeros = jnp.zeros((16,), jnp.float32)

        def zero_body(g, c):
            ws_v[pl.ds(g * 16, 16)] = fzeros
            return c

        lax.fori_loop(0, NSLOT // 16, zero_body, 0)

        def dest_body(g, c):
            v = eid_v[pl.ds(g * 16, 16)]
            d = plsc.load_gather(po_v, (v,)) + rank_v[pl.ds(g * 16, 16)]
            tok = (_iota16() + g * 16) >> 1
            plsc.store_scatter(src_v, (d,), tok)
            plsc.store_scatter(ws_v, (d,), wp_v[pl.ds(g * 16, 16)])
            return c

        lax.fori_loop(0, P // 16, dest_body, 0)

        pltpu.sync_copy(src_v, src_hbm)
        pltpu.sync_copy(ws_v, ws_hbm)
        pltpu.sync_copy(be_v, be_hbm)
        pltpu.sync_copy(bv_v, bv_hbm)


def _ffn_body(be_ref, bv_ref, srcc_ref, srcr_ref, wsc_ref, x_ref,
              w1_ref, w3_ref, w2_ref, out_ref):
    b = pl.program_id(0)

    @pl.when(b == 0)
    def _():
        out_ref[...] = jnp.zeros_like(out_ref)

    @pl.when(bv_ref[b] != 0)
    def _():
        tcol = srcc_ref[pl.ds(b * BLK, BLK), :]
        iota1 = lax.broadcasted_iota(jnp.int32, (BLK, S), 1)
        sel = (tcol == iota1).astype(jnp.bfloat16)
        xb = jnp.dot(sel, x_ref[...], preferred_element_type=jnp.float32)
        xb = xb.astype(jnp.bfloat16)
        g = jnp.dot(xb, w1_ref[0].astype(jnp.bfloat16),
                    preferred_element_type=jnp.float32)
        u = jnp.dot(xb, w3_ref[0].astype(jnp.bfloat16),
                    preferred_element_type=jnp.float32)
        act = g * (1.0 / (1.0 + jnp.exp(-g)))
        part = jnp.dot((act * u).astype(jnp.bfloat16),
                       w2_ref[0].astype(jnp.bfloat16),
                       preferred_element_type=jnp.float32)
        yw = (part * wsc_ref[pl.ds(b * BLK, BLK), :]).astype(jnp.bfloat16)
        srow = srcr_ref[b]
        iota0 = lax.broadcasted_iota(jnp.int32, (S, BLK), 0)
        selT = (iota0 == srow).astype(jnp.bfloat16)
        out_ref[...] += jnp.dot(selT, yw, preferred_element_type=jnp.float32)


def _ffn(be, bv, src_col, src_row, ws_col, x, w1, w3, w2):
    grid_spec = pltpu.PrefetchScalarGridSpec(
        num_scalar_prefetch=2,
        grid=(G,),
        in_specs=[
            pl.BlockSpec((NSLOT, 1), lambda b, be, bv: (0, 0)),
            pl.BlockSpec((G, 1, BLK), lambda b, be, bv: (0, 0, 0)),
            pl.BlockSpec((NSLOT, 1), lambda b, be, bv: (0, 0)),
            pl.BlockSpec((S, H), lambda b, be, bv: (0, 0)),
            pl.BlockSpec((1, H, F), lambda b, be, bv: (be[b], 0, 0)),
            pl.BlockSpec((1, H, F), lambda b, be, bv: (be[b], 0, 0)),
            pl.BlockSpec((1, F, H), lambda b, be, bv: (be[b], 0, 0)),
        ],
        out_specs=pl.BlockSpec((S, H), lambda b, be, bv: (0, 0)),
    )
    x = x.astype(jnp.bfloat16)
    return pl.pallas_call(
        _ffn_body,
        grid_spec=grid_spec,
        out_shape=jax.ShapeDtypeStruct((S, H), jnp.float32),
        compiler_params=pltpu.CompilerParams(vmem_limit_bytes=117 * 1024 * 1024),
    )(be, bv, src_col, src_row, ws_col, x, w1, w3, w2)


def kernel(hidden_states, gate_w, w1, w2, w3):
    b, s, h = hidden_states.shape
    x = hidden_states.reshape(s, h)
    topi, topw = _router(x, gate_w)
    src, ws, be, bv = _plan(topi.reshape(-1), topw.reshape(-1))
    out = _ffn(be, bv, src.reshape(NSLOT, 1), src.reshape(G, 1, BLK),
               ws.reshape(NSLOT, 1), x, w1, w3, w2)
    return out.reshape(b, s, h)


# speedup vs baseline: 1.0471x; 1.0089x over previous
import functools

import jax
import jax.numpy as jnp
from jax import lax
from jax.experimental import pallas as pl
from jax.experimental.pallas import tpu as pltpu
from jax.experimental.pallas import tpu_sc as plsc

H = 768
F = 1024
E = 64
K = 2
S = 2048
P = S * K
BLK = 128
LOG2BLK = 7
G = 96
NSLOT = G * BLK
NEG = -1e30

_MESH = plsc.VectorSubcoreMesh(core_axis_name="c", subcore_axis_name="s")
_SC_PARAMS = pltpu.CompilerParams(needs_layout_passes=False)


def _router_body(x_ref, gw_ref, topi_ref, topw_ref):
    logits = jnp.dot(x_ref[...], gw_ref[...], preferred_element_type=jnp.float32)
    idx = lax.broadcasted_iota(jnp.int32, (S, E), 1)
    m1 = jnp.max(logits, axis=1, keepdims=True)
    a1 = jnp.min(jnp.where(logits >= m1, idx, E), axis=1, keepdims=True)
    l2 = jnp.where(idx == a1, NEG, logits)
    m2 = jnp.max(l2, axis=1, keepdims=True)
    a2 = jnp.min(jnp.where((l2 >= m2) & (idx != a1), idx, E), axis=1, keepdims=True)
    e2 = jnp.exp(m2 - m1)
    w1 = 1.0 / (1.0 + e2)
    topi_ref[...] = jnp.concatenate([a1, a2], axis=1)
    topw_ref[...] = jnp.concatenate([w1, 1.0 - w1], axis=1)


def _router(x, gate_w):
    return pl.pallas_call(
        _router_body,
        out_shape=(
            jax.ShapeDtypeStruct((S, K), jnp.int32),
            jax.ShapeDtypeStruct((S, K), jnp.float32),
        ),
    )(x, gate_w)


def _iota16():
    return lax.broadcasted_iota(jnp.int32, (16,), 0)


@functools.partial(
    pl.kernel,
    mesh=_MESH,
    out_type=(
        jax.ShapeDtypeStruct((NSLOT,), jnp.int32),
        jax.ShapeDtypeStruct((NSLOT,), jnp.float32),
        jax.ShapeDtypeStruct((G,), jnp.int32),
        jax.ShapeDtypeStruct((G,), jnp.int32),
    ),
    scratch_types=(
        pltpu.VMEM((P,), jnp.int32),
        pltpu.VMEM((P,), jnp.int32),
        pltpu.VMEM((E,), jnp.int32),
        pltpu.VMEM((E,), jnp.int32),
        pltpu.VMEM((112,), jnp.int32),
        pltpu.VMEM((NSLOT,), jnp.int32),
        pltpu.VMEM((P,), jnp.float32),
        pltpu.VMEM((NSLOT,), jnp.float32),
        pltpu.VMEM((G,), jnp.int32),
        pltpu.VMEM((G,), jnp.int32),
    ),
    compiler_params=_SC_PARAMS,
)
def _plan(eid_hbm, wp_hbm, src_hbm, ws_hbm, be_hbm, bv_hbm,
          eid_v, rank_v, cnt_v, po_v, eh_v, src_v, wp_v, ws_v, be_v, bv_v):
    wid = lax.axis_index("s") * 2 + lax.axis_index("c")

    @pl.when(wid == 0)
    def _():
        pltpu.sync_copy(eid_hbm, eid_v)
        pltpu.sync_copy(wp_hbm, wp_v)
        zeros = jnp.zeros((16,), jnp.int32)
        for g in range(E // 16):
            cnt_v[pl.ds(g * 16, 16)] = zeros

        def rank_body(g, c):
            for u in range(2):
                o = g * 32 + u * 16
                v = eid_v[pl.ds(o, 16)]
                base = plsc.load_gather(cnt_v, (v,))
                dup, lastm = plsc.scan_count(v)
                rank_v[pl.ds(o, 16)] = base + dup - 1
                plsc.store_scatter(cnt_v, (v,), base + dup, mask=lastm)
            return c

        lax.fori_loop(0, P // 32, rank_body, 0)

        for g in range(7):
            eh_v[pl.ds(g * 16, 16)] = zeros
        ones = jnp.ones((16,), jnp.int32)
        carry = jnp.int32(0)
        e_last = jnp.int32(0)
        for g in range(E // 16):
            cnt = cnt_v[pl.ds(g * 16, 16)]
            nb = (cnt + (BLK - 1)) >> LOG2BLK
            incl = plsc.cumsum(nb)
            excl = carry + incl - nb
            po_v[pl.ds(g * 16, 16)] = excl * BLK
            plsc.addupdate_scatter(eh_v, (excl + nb,), ones)
            carry = carry + jnp.max(incl, axis=0)
            gid = _iota16() + g * 16
            e_last = jnp.maximum(e_last, jnp.max(jnp.where(cnt > 0, gid, -1), axis=0))
        nblocks = carry

        bcarry = jnp.int32(0)
        for g in range(G // 16):
            h = eh_v[pl.ds(g * 16, 16)]
            c = plsc.cumsum(h) + bcarry
            bid = _iota16() + g * 16
            valid = bid < nblocks
            be_v[pl.ds(g * 16, 16)] = jnp.where(valid, jnp.minimum(c, E - 1), e_last)
            bv_v[pl.ds(g * 16, 16)] = jnp.where(valid, 1, 0)
            bcarry = jnp.max(c, axis=0)

        fzeros = jnp.zeros((16,), jnp.float32)

        def zero_body(g, c):
            for u in range(4):
                ws_v[pl.ds(g * 64 + u * 16, 16)] = fzeros
            return c

        lax.fori_loop(0, NSLOT // 64, zero_body, 0)

        def dest_body(g, c):
            for u in range(2):
                o = g * 32 + u * 16
                v = eid_v[pl.ds(o, 16)]
                d = plsc.load_gather(po_v, (v,)) + rank_v[pl.ds(o, 16)]
                tok = (_iota16() + o) >> 1
                plsc.store_scatter(src_v, (d,), tok)
                plsc.store_scatter(ws_v, (d,), wp_v[pl.ds(o, 16)])
            return c

        lax.fori_loop(0, P // 32, dest_body, 0)

        pltpu.sync_copy(src_v, src_hbm)
        pltpu.sync_copy(ws_v, ws_hbm)
        pltpu.sync_copy(be_v, be_hbm)
        pltpu.sync_copy(bv_v, bv_hbm)


def _ffn_body(be_ref, bv_ref, srcc_ref, srcr_ref, wsc_ref, x_ref,
              w1_ref, w3_ref, w2_ref, out_ref):
    b = pl.program_id(0)

    @pl.when(b == 0)
    def _():
        out_ref[...] = jnp.zeros_like(out_ref)

    @pl.when(bv_ref[b] != 0)
    def _():
        tcol = srcc_ref[pl.ds(b * BLK, BLK), :]
        iota1 = lax.broadcasted_iota(jnp.int32, (BLK, S), 1)
        sel = (tcol == iota1).astype(jnp.bfloat16)
        xb = jnp.dot(sel, x_ref[...], preferred_element_type=jnp.float32)
        xb = xb.astype(jnp.bfloat16)
        g = jnp.dot(xb, w1_ref[0].astype(jnp.bfloat16),
                    preferred_element_type=jnp.float32)
        u = jnp.dot(xb, w3_ref[0].astype(jnp.bfloat16),
                    preferred_element_type=jnp.float32)
        act = g * (1.0 / (1.0 + jnp.exp(-g)))
        part = jnp.dot((act * u).astype(jnp.bfloat16),
                       w2_ref[0].astype(jnp.bfloat16),
                       preferred_element_type=jnp.float32)
        yw = (part * wsc_ref[pl.ds(b * BLK, BLK), :]).astype(jnp.bfloat16)
        srow = srcr_ref[b]
        iota0 = lax.broadcasted_iota(jnp.int32, (S, BLK), 0)
        selT = (iota0 == srow).astype(jnp.bfloat16)
        out_ref[...] += jnp.dot(selT, yw, preferred_element_type=jnp.float32)


def _ffn(be, bv, src_col, src_row, ws_col, x, w1, w3, w2):
    grid_spec = pltpu.PrefetchScalarGridSpec(
        num_scalar_prefetch=2,
        grid=(G,),
        in_specs=[
            pl.BlockSpec((NSLOT, 1), lambda b, be, bv: (0, 0)),
            pl.BlockSpec((G, 1, BLK), lambda b, be, bv: (0, 0, 0)),
            pl.BlockSpec((NSLOT, 1), lambda b, be, bv: (0, 0)),
            pl.BlockSpec((S, H), lambda b, be, bv: (0, 0)),
            pl.BlockSpec((1, H, F), lambda b, be, bv: (be[b], 0, 0)),
            pl.BlockSpec((1, H, F), lambda b, be, bv: (be[b], 0, 0)),
            pl.BlockSpec((1, F, H), lambda b, be, bv: (be[b], 0, 0)),
        ],
        out_specs=pl.BlockSpec((S, H), lambda b, be, bv: (0, 0)),
    )
    x = x.astype(jnp.bfloat16)
    return pl.pallas_call(
        _ffn_body,
        grid_spec=grid_spec,
        out_shape=jax.ShapeDtypeStruct((S, H), jnp.float32),
        compiler_params=pltpu.CompilerParams(vmem_limit_bytes=117 * 1024 * 1024),
    )(be, bv, src_col, src_row, ws_col, x, w1, w3, w2)


def kernel(hidden_states, gate_w, w1, w2, w3):
    b, s, h = hidden_states.shape
    x = hidden_states.reshape(s, h)
    topi, topw = _router(x, gate_w)
    src, ws, be, bv = _plan(topi.reshape(-1), topw.reshape(-1))
    out = _ffn(be, bv, src.reshape(NSLOT, 1), src.reshape(G, 1, BLK),
               ws.reshape(NSLOT, 1), x, w1, w3, w2)
    return out.reshape(b, s, h)
